# Initial kernel scaffold; baseline (speedup 1.0000x reference)
import jax
import jax.numpy as jnp
from jax.experimental import pallas as pl


def kernel(hidden_states, memory_tokens, memory_mask):
    raise NotImplementedError("write your pallas kernel here")


# speedup vs baseline: 1.3571x; 1.3571x over previous
import functools
import math

import jax
import jax.numpy as jnp
from jax.experimental import pallas as pl


def _attn_body(h_ref, mem_ref, mask_ref, out_ref, *, scale):
    q = h_ref[0]
    k = mem_ref[0]
    m = mask_ref[0]
    scores = jax.lax.dot_general(
        q, k, (((1,), (1,)), ((), ())), preferred_element_type=jnp.float32
    ) * scale
    scores = jnp.where(m > 0.0, scores, jnp.float32(-1e9))
    mx = jnp.max(scores, axis=-1, keepdims=True)
    w = jnp.exp(scores - mx)
    denom = jnp.sum(w, axis=-1, keepdims=True)
    wn = w / denom
    fused = jax.lax.dot_general(
        wn, k, (((1,), (0,)), ((), ())), preferred_element_type=jnp.float32
    )
    out = fused - q
    row_valid = jnp.max(m) > 0.0
    out_ref[0] = jnp.where(row_valid, out, jnp.zeros_like(out))


def kernel(hidden_states, memory_tokens, memory_mask):
    B, S, D = hidden_states.shape
    M = memory_tokens.shape[1]
    mask_f = memory_mask.reshape(B, 1, M).astype(jnp.float32)
    scale = 1.0 / math.sqrt(D)
    return pl.pallas_call(
        functools.partial(_attn_body, scale=scale),
        grid=(B,),
        in_specs=[
            pl.BlockSpec((1, S, D), lambda b: (b, 0, 0)),
            pl.BlockSpec((1, M, D), lambda b: (b, 0, 0)),
            pl.BlockSpec((1, 1, M), lambda b: (b, 0, 0)),
        ],
        out_specs=pl.BlockSpec((1, S, D), lambda b: (b, 0, 0)),
        out_shape=jax.ShapeDtypeStruct((B, S, D), jnp.float32),
    )(hidden_states, memory_tokens, mask_f)


# speedup vs baseline: 1.5986x; 1.1779x over previous
import functools
import math

import jax
import jax.numpy as jnp
from jax.experimental import pallas as pl


def _attn_body(h_ref, mem_ref, mask_ref, out_ref, *, scale):
    q = h_ref[0]
    k = mem_ref[0]
    m = mask_ref[0]
    qs = q * scale
    scores = jax.lax.dot_general(
        qs, k, (((1,), (1,)), ((), ())), preferred_element_type=jnp.float32
    )
    scores = jnp.where(m > 0.0, scores, jnp.float32(-1e9))
    w = jnp.exp(scores)
    fused_un = jax.lax.dot_general(
        w, k, (((1,), (0,)), ((), ())), preferred_element_type=jnp.float32
    )
    denom = jnp.sum(w, axis=-1, keepdims=True)
    out = fused_un * (1.0 / denom) - q
    row_valid = jnp.max(m) > 0.0
    out_ref[0] = jnp.where(row_valid, out, jnp.zeros_like(out))


def kernel(hidden_states, memory_tokens, memory_mask):
    B, S, D = hidden_states.shape
    M = memory_tokens.shape[1]
    mask_f = memory_mask.reshape(B, 1, M).astype(jnp.float32)
    scale = 1.0 / math.sqrt(D)
    return pl.pallas_call(
        functools.partial(_attn_body, scale=scale),
        grid=(B,),
        in_specs=[
            pl.BlockSpec((1, S, D), lambda b: (b, 0, 0)),
            pl.BlockSpec((1, M, D), lambda b: (b, 0, 0)),
            pl.BlockSpec((1, 1, M), lambda b: (b, 0, 0)),
        ],
        out_specs=pl.BlockSpec((1, S, D), lambda b: (b, 0, 0)),
        out_shape=jax.ShapeDtypeStruct((B, S, D), jnp.float32),
    )(hidden_states, memory_tokens, mask_f)
